# Initial kernel scaffold; baseline (speedup 1.0000x reference)
import jax
import jax.numpy as jnp
from jax.experimental import pallas as pl


def kernel(x, edge_index, Wl1, Wr1, b1, g1, be1, Wl2, Wr2, b2, g2, be2, Wl3, Wr3, b3):
    raise NotImplementedError("write your pallas kernel here")


# speedup vs baseline: 3.8220x; 3.8220x over previous
import functools

import jax
import jax.numpy as jnp
from jax import lax
from jax.experimental import pallas as pl
from jax.experimental.pallas import tpu as pltpu
from jax.experimental.pallas import tpu_sc as plsc

N = 10000
E = 320000
D = 128
H = 128

NC = 2
NS = 16
NW = NC * NS
CH = 128
CHUNKS_PER_W = -(-E // (NW * CH))
EW = CHUNKS_PER_W * CH
EPAD = EW * NW
RA = 632
NPAD = RA * NS
ZOFF = (0, 128, 256, 384, 504)


def _seg_sum_sc():
    mesh = plsc.VectorSubcoreMesh(core_axis_name="c", subcore_axis_name="s")
    out_type = [jax.ShapeDtypeStruct((NC, NPAD, H), jnp.float32),
                jax.ShapeDtypeStruct((NC, NPAD, H), jnp.float32)]
    scratch = [
        pltpu.VMEM((CH,), jnp.int32),
        pltpu.VMEM((1, CH), jnp.int32),
        pltpu.VMEM((CH, H), jnp.float32),
        pltpu.VMEM_SHARED((NPAD, H), jnp.float32),
        pltpu.SemaphoreType.DMA,
    ]

    @functools.partial(pl.kernel, mesh=mesh, out_type=out_type,
                       scratch_types=scratch)
    def seg_kernel(tab_hbm, src_hbm, dst_hbm, dep_hbm, before_hbm, after_hbm,
                   src_v, dst_v, rows_v, acc_sh, sem):
        del dep_hbm
        cid = lax.axis_index("c")
        sid = lax.axis_index("s")
        wid = sid * NC + cid
        row0 = pl.multiple_of(sid * RA, 8)

        pltpu.sync_copy(acc_sh.at[pl.ds(row0, RA)],
                        before_hbm.at[cid, pl.ds(row0, RA)])

        plsc.subcore_barrier()

        base0 = wid * EW

        @pl.loop(0, CHUNKS_PER_W)
        def _(i):
            base = base0 + i * CH
            pltpu.sync_copy(src_hbm.at[pl.ds(base, CH)], src_v)
            pltpu.sync_copy(dst_hbm.at[pl.ds(base, CH)], dst_v.at[0])
            pltpu.async_copy(tab_hbm.at[src_v], rows_v, sem).wait()
            pltpu.sync_copy(rows_v, acc_sh.at[dst_v.at[0]], add=True)

        plsc.subcore_barrier()

        pltpu.sync_copy(acc_sh.at[pl.ds(row0, RA)],
                        after_hbm.at[cid, pl.ds(row0, RA)])

    return seg_kernel


def _cnt_sc():
    mesh = plsc.VectorSubcoreMesh(core_axis_name="c", subcore_axis_name="s")
    out_type = [jax.ShapeDtypeStruct((NC, NPAD, 16), jnp.float32),
                jax.ShapeDtypeStruct((NC, NPAD, 16), jnp.float32)]
    scratch = [
        pltpu.VMEM((1, CH), jnp.int32),
        pltpu.VMEM((CH, 16), jnp.float32),
        pltpu.VMEM_SHARED((NPAD, 16), jnp.float32),
    ]

    @functools.partial(pl.kernel, mesh=mesh, out_type=out_type,
                       scratch_types=scratch)
    def cnt_kernel(dst_hbm, ones_hbm, before_hbm, after_hbm,
                   dst_v, ones_v, cnt_sh):
        cid = lax.axis_index("c")
        sid = lax.axis_index("s")
        wid = sid * NC + cid
        row0 = pl.multiple_of(sid * RA, 8)

        pltpu.sync_copy(ones_hbm, ones_v)
        pltpu.sync_copy(cnt_sh.at[pl.ds(row0, RA)],
                        before_hbm.at[cid, pl.ds(row0, RA)])

        plsc.subcore_barrier()

        base0 = wid * EW

        @pl.loop(0, CHUNKS_PER_W)
        def _(i):
            base = base0 + i * CH
            pltpu.sync_copy(dst_hbm.at[pl.ds(base, CH)], dst_v.at[0])
            pltpu.sync_copy(ones_v, cnt_sh.at[dst_v.at[0]], add=True)

        plsc.subcore_barrier()

        pltpu.sync_copy(cnt_sh.at[pl.ds(row0, RA)],
                        after_hbm.at[cid, pl.ds(row0, RA)])

    return cnt_kernel


@functools.cache
def _seg_sum_sc_cached():
    return _seg_sum_sc()


@functools.cache
def _cnt_sc_cached():
    return _cnt_sc()


DIFF_BLK = 1264


def _diff_body(sa_ref, sb_ref, o_ref):
    o_ref[...] = (sa_ref[0] - sb_ref[0]) + (sa_ref[1] - sb_ref[1])


def _tc_diff(sa, sb, width):
    spec = pl.BlockSpec((NC, DIFF_BLK, width), lambda i: (0, i, 0))
    return pl.pallas_call(
        _diff_body,
        grid=(NPAD // DIFF_BLK,),
        in_specs=[spec, spec],
        out_specs=pl.BlockSpec((DIFF_BLK, width), lambda i: (i, 0)),
        out_shape=jax.ShapeDtypeStruct((NPAD, width), jnp.float32),
    )(sa, sb)

def _proj_body(x_ref, wl_ref, wr_ref, p_ref, q_ref):
    xv = x_ref[...]
    p_ref[...] = jnp.dot(xv, wl_ref[...], preferred_element_type=jnp.float32)
    q_ref[...] = jnp.dot(xv, wr_ref[...], preferred_element_type=jnp.float32)


def _tc_proj(x, wl, wr):
    return pl.pallas_call(
        _proj_body,
        out_shape=[jax.ShapeDtypeStruct((N, H), jnp.float32),
                   jax.ShapeDtypeStruct((N, H), jnp.float32)],
    )(x, wl, wr)


def _combine1_body(s_ref, c_ref, q_ref, b_ref, g_ref,
                   be_ref, wl_ref, wr_ref, p2_ref, q2_ref, inv_ref):
    cnt = c_ref[:, 0:1]
    inv = 1.0 / jnp.maximum(cnt, 1.0)
    h = s_ref[...] * inv + q_ref[...] + b_ref[...]
    mu = jnp.mean(h, axis=0, keepdims=True)
    var = jnp.mean((h - mu) ** 2, axis=0, keepdims=True)
    h = (h - mu) / jnp.sqrt(var + 1e-5) * g_ref[...] + be_ref[...]
    h = jnp.maximum(h, 0.0)
    p2_ref[...] = jnp.dot(h, wl_ref[...], preferred_element_type=jnp.float32)
    q2_ref[...] = jnp.dot(h, wr_ref[...], preferred_element_type=jnp.float32)
    inv_ref[...] = inv


def _tc_combine1(s, c, q, b, g, be, wl, wr):
    return pl.pallas_call(
        _combine1_body,
        out_shape=[jax.ShapeDtypeStruct((N, H), jnp.float32),
                   jax.ShapeDtypeStruct((N, H), jnp.float32),
                   jax.ShapeDtypeStruct((N, 1), jnp.float32)],
    )(s, c, q, b, g, be, wl, wr)


def _combine2_body(s_ref, inv_ref, q_ref, b_ref, g_ref, be_ref,
                   wl_ref, wr_ref, p3_ref, q3_ref):
    inv = inv_ref[...]
    h = s_ref[...] * inv + q_ref[...] + b_ref[...]
    mu = jnp.mean(h, axis=0, keepdims=True)
    var = jnp.mean((h - mu) ** 2, axis=0, keepdims=True)
    h = (h - mu) / jnp.sqrt(var + 1e-5) * g_ref[...] + be_ref[...]
    h = jnp.maximum(h, 0.0)
    p3 = jnp.dot(h, wl_ref[...], preferred_element_type=jnp.float32)
    p3_ref[...] = jnp.broadcast_to(p3, (N, H))
    q3_ref[...] = jnp.dot(h, wr_ref[...], preferred_element_type=jnp.float32)


def _tc_combine2(s, inv, q, b, g, be, wl, wr):
    return pl.pallas_call(
        _combine2_body,
        out_shape=[jax.ShapeDtypeStruct((N, H), jnp.float32),
                   jax.ShapeDtypeStruct((N, 1), jnp.float32)],
    )(s, inv, q, b, g, be, wl, wr)


def _final_body(s_ref, inv_ref, q_ref, b_ref, out_ref):
    out_ref[...] = s_ref[:, 0:1] * inv_ref[...] + q_ref[...] + b_ref[...]


def _tc_final(s, inv, q, b):
    return pl.pallas_call(
        _final_body,
        out_shape=jax.ShapeDtypeStruct((N, 1), jnp.float32),
    )(s, inv, q, b)


def kernel(x, edge_index, Wl1, Wr1, b1, g1, be1, Wl2, Wr2, b2, g2, be2,
           Wl3, Wr3, b3):
    src = edge_index[0]
    dst = edge_index[1]
    pad = EPAD - E
    src_p = jnp.concatenate([src, jnp.zeros((pad,), jnp.int32)])
    dst_p = jnp.concatenate([dst, jnp.full((pad,), N, jnp.int32)])

    zeros_pad = jnp.zeros((NPAD, H), jnp.float32)
    zeros16 = jnp.zeros((NPAD, 16), jnp.float32)
    ones_ch = jnp.ones((CH, 16), jnp.float32)
    rid = jnp.arange(NPAD, dtype=jnp.int32)

    b1r = b1.reshape(1, H)
    g1r = g1.reshape(1, H)
    be1r = be1.reshape(1, H)
    b2r = b2.reshape(1, H)
    g2r = g2.reshape(1, H)
    be2r = be2.reshape(1, H)
    b3r = b3.reshape(1, 1)

    cb, ca = _cnt_sc_cached()(dst_p, ones_ch)
    c = _tc_diff(ca, cb, 16)
    p1, q1 = _tc_proj(x, Wl1, Wr1)
    sb1, sa1 = _seg_sum_sc_cached()(p1, src_p, dst_p, ca[:, :1, :1])
    s1 = _tc_diff(sa1, sb1, H)
    p2, q2, inv = _tc_combine1(s1[:N], c[:N], q1, b1r, g1r, be1r, Wl2, Wr2)
    sb2, sa2 = _seg_sum_sc_cached()(p2, src_p, dst_p, sa1[:, :1, :1])
    s2 = _tc_diff(sa2, sb2, H)
    p3, q3 = _tc_combine2(s2[:N], inv, q2, b2r, g2r, be2r, Wl3, Wr3)
    sb3, sa3 = _seg_sum_sc_cached()(p3, src_p, dst_p, sa2[:, :1, :1])
    s3 = _tc_diff(sa3, sb3, H)
    out = _tc_final(s3[:N], inv, q3, b3r)
    return out.reshape(N)
